# baseline (speedup 1.0000x reference)
import functools

import jax
import jax.numpy as jnp
from jax.experimental import pallas as pl

TOKEN_BLOCK = 2048


def _gate_block(x_ref, wt_ref, o_ref):
    logits = jax.lax.dot_general(
        x_ref[...],
        wt_ref[...],
        dimension_numbers=(((1,), (0,)), ((), ())),
        preferred_element_type=jnp.float32,
        precision=jax.lax.Precision.DEFAULT,
    )
    o_ref[...] = (logits > 0.0).astype(jnp.int32)


@jax.jit
def kernel(cls_hidden_states, gate_w):
    tokens, hidden = cls_hidden_states.shape
    num_experts = gate_w.shape[0]
    wt = gate_w.T

    grid = (tokens // TOKEN_BLOCK,)
    return pl.pallas_call(
        _gate_block,
        grid=grid,
        in_specs=[
            pl.BlockSpec((TOKEN_BLOCK, hidden), lambda i: (i, 0)),
            pl.BlockSpec((hidden, num_experts), lambda i: (0, 0)),
        ],
        out_specs=pl.BlockSpec((TOKEN_BLOCK, num_experts), lambda i: (i, 0)),
        out_shape=jax.ShapeDtypeStruct((tokens, num_experts), jnp.int32),
    )(cls_hidden_states, wt)


# speedup vs baseline: 1.0412x; 1.0412x over previous
import functools

import jax
import jax.numpy as jnp
from jax.experimental import pallas as pl

TOKEN_BLOCK = 1024


def _gate_block(x_ref, wt_ref, o_ref):
    logits = jax.lax.dot_general(
        x_ref[...],
        wt_ref[...],
        dimension_numbers=(((1,), (0,)), ((), ())),
        preferred_element_type=jnp.float32,
        precision=jax.lax.Precision.DEFAULT,
    )
    o_ref[...] = (logits > 0.0).astype(jnp.int32)


@jax.jit
def kernel(cls_hidden_states, gate_w):
    tokens, hidden = cls_hidden_states.shape
    num_experts = gate_w.shape[0]
    wt = gate_w.T

    grid = (tokens // TOKEN_BLOCK,)
    return pl.pallas_call(
        _gate_block,
        grid=grid,
        in_specs=[
            pl.BlockSpec((TOKEN_BLOCK, hidden), lambda i: (i, 0)),
            pl.BlockSpec((hidden, num_experts), lambda i: (0, 0)),
        ],
        out_specs=pl.BlockSpec((TOKEN_BLOCK, num_experts), lambda i: (i, 0)),
        out_shape=jax.ShapeDtypeStruct((tokens, num_experts), jnp.int32),
    )(cls_hidden_states, wt)
